# baseline (speedup 1.0000x reference)
import functools
import math

import jax
import jax.numpy as jnp
from jax import lax
from jax.experimental import pallas as pl
from jax.experimental.pallas import tpu as pltpu

_SPARSE_RATIO = 0.6


def _body(sa_ref, m2_ref, m3_ref, tok_hbm, out_hbm, mask_ref,
          ibuf, obuf, maskT_ref, isem, osem,
          *, num_keep, blk_n, n, nbuf, nblocks, nbpb):
    def norm(s):
        mn = jnp.min(s, axis=-1, keepdims=True)
        mx = jnp.max(s, axis=-1, keepdims=True)
        return (s - mn) / (mx - mn + 1e-08)

    def fetch(s):
        b, jo = s // nbpb, s % nbpb
        pltpu.make_async_copy(
            tok_hbm.at[b, pl.ds(jo * blk_n, blk_n), :],
            ibuf.at[s % nbuf], isem.at[s % nbuf]).start()

    for s in range(nbuf):
        fetch(s)

    score = (norm(sa_ref[...]) + norm(m2_ref[...]) + norm(m3_ref[...])) / 3.0
    bits = lax.bitcast_convert_type(score, jnp.int32)
    nb = score.shape[0]
    lo0 = jnp.zeros((nb, 1), jnp.int32)
    hi0 = jnp.full((nb, 1), 0x3F800001, jnp.int32)

    def bis(_, carry):
        lo, hi = carry
        mid = lo + (hi - lo) // 2
        cnt = jnp.sum((bits >= mid).astype(jnp.int32), axis=-1, keepdims=True)
        ge = cnt >= num_keep
        return jnp.where(ge, mid, lo), jnp.where(ge, hi, mid)

    tbits, _ = lax.fori_loop(0, 30, bis, (lo0, hi0))
    gt = bits > tbits
    eq = bits == tbits
    need = num_keep - jnp.sum(gt.astype(jnp.int32), axis=-1, keepdims=True)
    idx = lax.broadcasted_iota(jnp.int32, score.shape, 1)
    eqcum = eq.astype(jnp.int32)
    d = 1
    while d < n:
        rolled = pltpu.roll(eqcum, d, axis=1)
        eqcum = eqcum + jnp.where(idx >= d, rolled, 0)
        d *= 2
    mask = (gt | (eq & (eqcum <= need))).astype(jnp.float32)
    mask_ref[...] = mask
    maskT_ref[...] = mask.T

    for s in range(nblocks):
        slot = s % nbuf
        b, jo = s // nbpb, s % nbpb
        if s >= nbuf:
            bp, jp = (s - nbuf) // nbpb, (s - nbuf) % nbpb
            pltpu.make_async_copy(
                obuf.at[slot],
                out_hbm.at[bp, pl.ds(jp * blk_n, blk_n), :],
                osem.at[slot]).wait()
        pltpu.make_async_copy(
            tok_hbm.at[b, pl.ds(jo * blk_n, blk_n), :],
            ibuf.at[slot], isem.at[slot]).wait()
        m = maskT_ref[pl.ds(jo * blk_n, blk_n), b:b + 1]
        obuf[slot] = ibuf[slot] * m
        pltpu.make_async_copy(
            obuf.at[slot],
            out_hbm.at[b, pl.ds(jo * blk_n, blk_n), :],
            osem.at[slot]).start()
        if s + nbuf < nblocks:
            fetch(s + nbuf)

    for s in range(max(0, nblocks - nbuf), nblocks):
        slot = s % nbuf
        b, jo = s // nbpb, s % nbpb
        pltpu.make_async_copy(
            obuf.at[slot],
            out_hbm.at[b, pl.ds(jo * blk_n, blk_n), :],
            osem.at[slot]).wait()


def kernel(tokens, self_attention, cross_attention_m2, cross_attention_m3):
    B, N, C = tokens.shape
    num_keep = max(1, math.ceil(N * _SPARSE_RATIO))
    blk_n = 512
    nbuf = 6
    nbpb = N // blk_n
    nblocks = B * nbpb
    body = functools.partial(_body, num_keep=num_keep, blk_n=blk_n, n=N,
                             nbuf=nbuf, nblocks=nblocks, nbpb=nbpb)
    masked, mask = pl.pallas_call(
        body,
        in_specs=[
            pl.BlockSpec(memory_space=pltpu.VMEM),
            pl.BlockSpec(memory_space=pltpu.VMEM),
            pl.BlockSpec(memory_space=pltpu.VMEM),
            pl.BlockSpec(memory_space=pl.ANY),
        ],
        out_specs=[
            pl.BlockSpec(memory_space=pl.ANY),
            pl.BlockSpec(memory_space=pltpu.VMEM),
        ],
        out_shape=[
            jax.ShapeDtypeStruct((B, N, C), tokens.dtype),
            jax.ShapeDtypeStruct((B, N), jnp.float32),
        ],
        scratch_shapes=[
            pltpu.VMEM((nbuf, blk_n, C), jnp.float32),
            pltpu.VMEM((nbuf, blk_n, C), jnp.float32),
            pltpu.VMEM((N, B), jnp.float32),
            pltpu.SemaphoreType.DMA((nbuf,)),
            pltpu.SemaphoreType.DMA((nbuf,)),
        ],
    )(self_attention, cross_attention_m2, cross_attention_m3, tokens)
    return masked, mask


# speedup vs baseline: 1.0279x; 1.0279x over previous
import functools
import math

import jax
import jax.numpy as jnp
from jax import lax
from jax.experimental import pallas as pl
from jax.experimental.pallas import tpu as pltpu

_SPARSE_RATIO = 0.6


def _body(sa_ref, m2_ref, m3_ref, tok_hbm, out_hbm, mask_ref,
          ibuf, obuf, maskT_ref, isem, osem,
          *, num_keep, blk_n, n, nbuf, nblocks, nbpb):
    def norm(s):
        mn = jnp.min(s, axis=-1, keepdims=True)
        mx = jnp.max(s, axis=-1, keepdims=True)
        return (s - mn) / (mx - mn + 1e-08)

    def fetch(s):
        b, jo = s // nbpb, s % nbpb
        pltpu.make_async_copy(
            tok_hbm.at[b, pl.ds(jo * blk_n, blk_n), :],
            ibuf.at[s % nbuf], isem.at[s % nbuf]).start()

    for s in range(nbuf):
        fetch(s)

    score = (norm(sa_ref[...]) + norm(m2_ref[...]) + norm(m3_ref[...])) / 3.0
    bits = lax.bitcast_convert_type(score, jnp.int32)
    nb = score.shape[0]
    lo0 = jnp.zeros((nb, 1), jnp.int32)
    hi0 = jnp.full((nb, 1), 0x3F800001, jnp.int32)

    def bis(_, carry):
        lo, hi = carry
        mid = lo + (hi - lo) // 2
        cnt = jnp.sum((bits >= mid).astype(jnp.int32), axis=-1, keepdims=True)
        ge = cnt >= num_keep
        return jnp.where(ge, mid, lo), jnp.where(ge, hi, mid)

    tbits, _ = lax.fori_loop(0, 30, bis, (lo0, hi0))
    gt = bits > tbits
    eq = bits == tbits
    need = num_keep - jnp.sum(gt.astype(jnp.int32), axis=-1, keepdims=True)
    idx = lax.broadcasted_iota(jnp.int32, score.shape, 1)
    eqcum = eq.astype(jnp.int32)
    d = 1
    while d < n:
        rolled = pltpu.roll(eqcum, d, axis=1)
        eqcum = eqcum + jnp.where(idx >= d, rolled, 0)
        d *= 2
    mask = (gt | (eq & (eqcum <= need))).astype(jnp.float32)
    mask_ref[...] = mask
    maskT_ref[...] = mask.T

    for s in range(nblocks):
        slot = s % nbuf
        b, jo = s // nbpb, s % nbpb
        if s >= nbuf:
            bp, jp = (s - nbuf) // nbpb, (s - nbuf) % nbpb
            pltpu.make_async_copy(
                obuf.at[slot],
                out_hbm.at[bp, pl.ds(jp * blk_n, blk_n), :],
                osem.at[slot]).wait()
        pltpu.make_async_copy(
            tok_hbm.at[b, pl.ds(jo * blk_n, blk_n), :],
            ibuf.at[slot], isem.at[slot]).wait()
        m = maskT_ref[pl.ds(jo * blk_n, blk_n), b:b + 1]
        obuf[slot] = ibuf[slot] * m
        pltpu.make_async_copy(
            obuf.at[slot],
            out_hbm.at[b, pl.ds(jo * blk_n, blk_n), :],
            osem.at[slot]).start()
        if s + nbuf < nblocks:
            fetch(s + nbuf)

    for s in range(max(0, nblocks - nbuf), nblocks):
        slot = s % nbuf
        b, jo = s // nbpb, s % nbpb
        pltpu.make_async_copy(
            obuf.at[slot],
            out_hbm.at[b, pl.ds(jo * blk_n, blk_n), :],
            osem.at[slot]).wait()


def kernel(tokens, self_attention, cross_attention_m2, cross_attention_m3):
    B, N, C = tokens.shape
    num_keep = max(1, math.ceil(N * _SPARSE_RATIO))
    blk_n = 1024
    nbuf = 4
    nbpb = N // blk_n
    nblocks = B * nbpb
    body = functools.partial(_body, num_keep=num_keep, blk_n=blk_n, n=N,
                             nbuf=nbuf, nblocks=nblocks, nbpb=nbpb)
    masked, mask = pl.pallas_call(
        body,
        in_specs=[
            pl.BlockSpec(memory_space=pltpu.VMEM),
            pl.BlockSpec(memory_space=pltpu.VMEM),
            pl.BlockSpec(memory_space=pltpu.VMEM),
            pl.BlockSpec(memory_space=pl.ANY),
        ],
        out_specs=[
            pl.BlockSpec(memory_space=pl.ANY),
            pl.BlockSpec(memory_space=pltpu.VMEM),
        ],
        out_shape=[
            jax.ShapeDtypeStruct((B, N, C), tokens.dtype),
            jax.ShapeDtypeStruct((B, N), jnp.float32),
        ],
        scratch_shapes=[
            pltpu.VMEM((nbuf, blk_n, C), jnp.float32),
            pltpu.VMEM((nbuf, blk_n, C), jnp.float32),
            pltpu.VMEM((N, B), jnp.float32),
            pltpu.SemaphoreType.DMA((nbuf,)),
            pltpu.SemaphoreType.DMA((nbuf,)),
        ],
    )(self_attention, cross_attention_m2, cross_attention_m3, tokens)
    return masked, mask


# speedup vs baseline: 1.0402x; 1.0119x over previous
import functools
import math

import jax
import jax.numpy as jnp
from jax import lax
from jax.experimental import pallas as pl
from jax.experimental.pallas import tpu as pltpu

_SPARSE_RATIO = 0.6


def _body(sa_ref, m2_ref, m3_ref, tok_hbm, out_hbm, mask_ref,
          ibuf, obuf, maskT_ref, isem, osem,
          *, num_keep, blk_n, n, nbuf, nblocks, nbpb):
    def norm(s):
        mn = jnp.min(s, axis=-1, keepdims=True)
        mx = jnp.max(s, axis=-1, keepdims=True)
        return (s - mn) / (mx - mn + 1e-08)

    def fetch(s):
        b, jo = s // nbpb, s % nbpb
        pltpu.make_async_copy(
            tok_hbm.at[b, pl.ds(jo * blk_n, blk_n), :],
            ibuf.at[s % nbuf], isem.at[s % nbuf]).start()

    for s in range(nbuf):
        fetch(s)

    score = (norm(sa_ref[...]) + norm(m2_ref[...]) + norm(m3_ref[...])) / 3.0
    bits = lax.bitcast_convert_type(score, jnp.int32)
    nb = score.shape[0]
    lo0 = jnp.zeros((nb, 1), jnp.int32)
    hi0 = jnp.full((nb, 1), 0x3F800001, jnp.int32)

    def bis(_, carry):
        lo, hi = carry
        mid = lo + (hi - lo) // 2
        cnt = jnp.sum((bits >= mid).astype(jnp.int32), axis=-1, keepdims=True)
        ge = cnt >= num_keep
        return jnp.where(ge, mid, lo), jnp.where(ge, hi, mid)

    tbits, _ = lax.fori_loop(0, 30, bis, (lo0, hi0))
    gt = bits > tbits
    eq = bits == tbits
    need = num_keep - jnp.sum(gt.astype(jnp.int32), axis=-1, keepdims=True)
    idx = lax.broadcasted_iota(jnp.int32, score.shape, 1)
    eqcum = eq.astype(jnp.int32)
    d = 1
    while d < n:
        rolled = pltpu.roll(eqcum, d, axis=1)
        eqcum = eqcum + jnp.where(idx >= d, rolled, 0)
        d *= 2
    mask = (gt | (eq & (eqcum <= need))).astype(jnp.float32)
    mask_ref[...] = mask
    maskT_ref[...] = mask.T

    for s in range(nblocks):
        slot = s % nbuf
        b, jo = s // nbpb, s % nbpb
        if s >= nbuf:
            bp, jp = (s - nbuf) // nbpb, (s - nbuf) % nbpb
            pltpu.make_async_copy(
                obuf.at[slot],
                out_hbm.at[bp, pl.ds(jp * blk_n, blk_n), :],
                osem.at[slot]).wait()
        pltpu.make_async_copy(
            tok_hbm.at[b, pl.ds(jo * blk_n, blk_n), :],
            ibuf.at[slot], isem.at[slot]).wait()
        m = maskT_ref[pl.ds(jo * blk_n, blk_n), b:b + 1]
        obuf[slot] = ibuf[slot] * m
        pltpu.make_async_copy(
            obuf.at[slot],
            out_hbm.at[b, pl.ds(jo * blk_n, blk_n), :],
            osem.at[slot]).start()
        if s + nbuf < nblocks:
            fetch(s + nbuf)

    for s in range(max(0, nblocks - nbuf), nblocks):
        slot = s % nbuf
        b, jo = s // nbpb, s % nbpb
        pltpu.make_async_copy(
            obuf.at[slot],
            out_hbm.at[b, pl.ds(jo * blk_n, blk_n), :],
            osem.at[slot]).wait()


def kernel(tokens, self_attention, cross_attention_m2, cross_attention_m3):
    B, N, C = tokens.shape
    num_keep = max(1, math.ceil(N * _SPARSE_RATIO))
    blk_n = 2048
    nbuf = 3
    nbpb = N // blk_n
    nblocks = B * nbpb
    body = functools.partial(_body, num_keep=num_keep, blk_n=blk_n, n=N,
                             nbuf=nbuf, nblocks=nblocks, nbpb=nbpb)
    masked, mask = pl.pallas_call(
        body,
        in_specs=[
            pl.BlockSpec(memory_space=pltpu.VMEM),
            pl.BlockSpec(memory_space=pltpu.VMEM),
            pl.BlockSpec(memory_space=pltpu.VMEM),
            pl.BlockSpec(memory_space=pl.ANY),
        ],
        out_specs=[
            pl.BlockSpec(memory_space=pl.ANY),
            pl.BlockSpec(memory_space=pltpu.VMEM),
        ],
        out_shape=[
            jax.ShapeDtypeStruct((B, N, C), tokens.dtype),
            jax.ShapeDtypeStruct((B, N), jnp.float32),
        ],
        scratch_shapes=[
            pltpu.VMEM((nbuf, blk_n, C), jnp.float32),
            pltpu.VMEM((nbuf, blk_n, C), jnp.float32),
            pltpu.VMEM((N, B), jnp.float32),
            pltpu.SemaphoreType.DMA((nbuf,)),
            pltpu.SemaphoreType.DMA((nbuf,)),
        ],
    )(self_attention, cross_attention_m2, cross_attention_m3, tokens)
    return masked, mask
